# baseline (speedup 1.0000x reference)
import functools

import jax
import jax.numpy as jnp
from jax.experimental import pallas as pl
from jax.experimental.pallas import tpu as pltpu


def _pass1_kernel(adj_hbm, x_ref, w1_ref, b1_ref, h_ref, adj16_hbm,
                  buf, st16, sems, semo, *, bm, slots):
    i = pl.program_id(0)
    nsteps = pl.num_programs(0)

    def in_copy(step, slot):
        return pltpu.make_async_copy(
            adj_hbm.at[pl.ds((step % nsteps) * bm, bm), :],
            buf.at[slot], sems.at[slot])

    def out_copy(step):
        return pltpu.make_async_copy(
            st16.at[0],
            adj16_hbm.at[pl.ds((step % nsteps) * bm, bm), :],
            semo.at[0])

    @pl.when(i == 0)
    def _():
        for s in range(slots - 1):
            in_copy(s, s).start()

    @pl.when(i + slots - 1 < nsteps)
    def _():
        in_copy(i + slots - 1, (i + slots - 1) % slots).start()

    slot = i % slots
    in_copy(i, slot).wait()

    a16 = buf[slot].astype(jnp.bfloat16)

    @pl.when(i >= 1)
    def _():
        out_copy(i - 1).wait()

    st16[0] = a16
    out_copy(i).start()

    t = jnp.dot(a16, x_ref[...], preferred_element_type=jnp.float32)
    t = jnp.dot(t.astype(jnp.bfloat16), w1_ref[...].astype(jnp.bfloat16),
                preferred_element_type=jnp.float32) + b1_ref[...]
    h_ref[...] = jnp.maximum(t, 0.0).astype(jnp.bfloat16)

    @pl.when(i == nsteps - 1)
    def _():
        out_copy(nsteps - 1).wait()


def _pass2_kernel(adj16_hbm, h_ref, w2_ref, b2_ref, out_ref, buf, sems,
                  *, bm, slots):
    i = pl.program_id(0)
    nsteps = pl.num_programs(0)

    def in_copy(step, slot):
        return pltpu.make_async_copy(
            adj16_hbm.at[pl.ds((step % nsteps) * bm, bm), :],
            buf.at[slot], sems.at[slot])

    @pl.when(i == 0)
    def _():
        for s in range(slots - 1):
            in_copy(s, s).start()

    @pl.when(i + slots - 1 < nsteps)
    def _():
        in_copy(i + slots - 1, (i + slots - 1) % slots).start()

    slot = i % slots
    in_copy(i, slot).wait()

    t = jnp.dot(buf[slot], h_ref[...], preferred_element_type=jnp.float32)
    t = jnp.dot(t.astype(jnp.bfloat16), w2_ref[...].astype(jnp.bfloat16),
                preferred_element_type=jnp.float32) + b2_ref[...]
    out_ref[...] = t


def _gcn(x, adj, W1, b1, W2, b2, *, bm1, bm2, slots):
    n, k = adj.shape
    d = W1.shape[1]
    x16 = x.astype(jnp.bfloat16)
    h16, adj16 = pl.pallas_call(
        functools.partial(_pass1_kernel, bm=bm1, slots=slots),
        grid=(n // bm1,),
        in_specs=[
            pl.BlockSpec(memory_space=pl.ANY),
            pl.BlockSpec((k, d), lambda i: (0, 0)),
            pl.BlockSpec((d, d), lambda i: (0, 0)),
            pl.BlockSpec((1, d), lambda i: (0, 0)),
        ],
        out_specs=[
            pl.BlockSpec((bm1, d), lambda i: (i, 0)),
            pl.BlockSpec(memory_space=pl.ANY),
        ],
        out_shape=[
            jax.ShapeDtypeStruct((n, d), jnp.bfloat16),
            jax.ShapeDtypeStruct((n, k), jnp.bfloat16),
        ],
        scratch_shapes=[
            pltpu.VMEM((slots, bm1, k), jnp.float32),
            pltpu.VMEM((1, bm1, k), jnp.bfloat16),
            pltpu.SemaphoreType.DMA((slots,)),
            pltpu.SemaphoreType.DMA((1,)),
        ],
        compiler_params=pltpu.CompilerParams(
            vmem_limit_bytes=63 * 1024 * 1024),
    )(adj, x16, W1, b1.reshape(1, d))
    return pl.pallas_call(
        functools.partial(_pass2_kernel, bm=bm2, slots=slots),
        grid=(n // bm2,),
        in_specs=[
            pl.BlockSpec(memory_space=pl.ANY),
            pl.BlockSpec((k, d), lambda i: (0, 0)),
            pl.BlockSpec((d, d), lambda i: (0, 0)),
            pl.BlockSpec((1, d), lambda i: (0, 0)),
        ],
        out_specs=pl.BlockSpec((bm2, d), lambda i: (i, 0)),
        out_shape=jax.ShapeDtypeStruct((n, d), jnp.float32),
        scratch_shapes=[
            pltpu.VMEM((slots, bm2, k), jnp.bfloat16),
            pltpu.SemaphoreType.DMA((slots,)),
        ],
        compiler_params=pltpu.CompilerParams(
            vmem_limit_bytes=63 * 1024 * 1024),
    )(adj16, h16, W2, b2.reshape(1, d))


def kernel(x, adj, W1, b1, W2, b2):
    return _gcn(x, adj, W1, b1, W2, b2, bm1=400, bm2=1000, slots=2)


# speedup vs baseline: 1.1275x; 1.1275x over previous
import functools

import jax
import jax.numpy as jnp
from jax.experimental import pallas as pl
from jax.experimental.pallas import tpu as pltpu


def _gcn_kernel(adj_hbm, x_hbm, w1_ref, b1_ref, w2_ref, b2_ref, out_ref,
                buf, h_ref, xst_ref, x16_ref, sems, semx,
                *, bm, nchunk, slots):
    p = pl.program_id(0)
    i = pl.program_id(1)
    nsteps = pl.num_programs(1)
    g = p * nsteps + i
    ck = bm // nchunk

    def issue(step, slot):
        base = (step % nsteps) * bm
        for c in range(nchunk):
            pltpu.make_async_copy(
                adj_hbm.at[pl.ds(base + c * ck, ck), :],
                buf.at[slot, pl.ds(c * ck, ck), :],
                sems.at[slot],
            ).start()

    @pl.when(g == 0)
    def _():
        xcopy = pltpu.make_async_copy(x_hbm, xst_ref, semx)
        xcopy.start()
        for s in range(slots - 1):
            issue(s, s)
        xcopy.wait()
        x16_ref[...] = xst_ref[...].astype(jnp.bfloat16)

    @pl.when(g + slots - 1 < 2 * nsteps)
    def _():
        issue(g + slots - 1, (g + slots - 1) % slots)

    slot = g % slots
    for c in range(nchunk):
        pltpu.make_async_copy(
            adj_hbm.at[pl.ds(c * ck, ck), :],
            buf.at[slot, pl.ds(c * ck, ck), :],
            sems.at[slot],
        ).wait()

    a16 = buf[slot].astype(jnp.bfloat16)

    @pl.when(p == 0)
    def _():
        t = jnp.dot(a16, x16_ref[...], preferred_element_type=jnp.float32)
        t = jnp.dot(t.astype(jnp.bfloat16), w1_ref[...].astype(jnp.bfloat16),
                    preferred_element_type=jnp.float32) + b1_ref[...]
        h_ref[pl.ds(i * bm, bm), :] = jnp.maximum(t, 0.0).astype(jnp.bfloat16)

    @pl.when(p == 1)
    def _():
        t = jnp.dot(a16, h_ref[...], preferred_element_type=jnp.float32)
        t = jnp.dot(t.astype(jnp.bfloat16), w2_ref[...].astype(jnp.bfloat16),
                    preferred_element_type=jnp.float32) + b2_ref[...]
        out_ref[...] = t


def _gcn(x, adj, W1, b1, W2, b2, *, bm, nchunk, slots):
    n, k = adj.shape
    d = W1.shape[1]
    return pl.pallas_call(
        functools.partial(_gcn_kernel, bm=bm, nchunk=nchunk, slots=slots),
        grid=(2, n // bm),
        in_specs=[
            pl.BlockSpec(memory_space=pl.ANY),
            pl.BlockSpec(memory_space=pl.ANY),
            pl.BlockSpec((d, d), lambda p, i: (0, 0)),
            pl.BlockSpec((1, d), lambda p, i: (0, 0)),
            pl.BlockSpec((d, d), lambda p, i: (0, 0)),
            pl.BlockSpec((1, d), lambda p, i: (0, 0)),
        ],
        out_specs=pl.BlockSpec((bm, d), lambda p, i: (p * i, 0)),
        out_shape=jax.ShapeDtypeStruct((n, d), jnp.float32),
        compiler_params=pltpu.CompilerParams(
            vmem_limit_bytes=63 * 1024 * 1024),
        scratch_shapes=[
            pltpu.VMEM((slots, bm, k), jnp.float32),
            pltpu.VMEM((n, d), jnp.bfloat16),
            pltpu.VMEM((k, d), jnp.float32),
            pltpu.VMEM((k, d), jnp.bfloat16),
            pltpu.SemaphoreType.DMA((slots,)),
            pltpu.SemaphoreType.DMA(()),
        ],
    )(adj, x, W1, b1.reshape(1, d), W2, b2.reshape(1, d))


def kernel(x, adj, W1, b1, W2, b2):
    return _gcn(x, adj, W1, b1, W2, b2, bm=400, nchunk=1, slots=2)


# speedup vs baseline: 1.1354x; 1.0070x over previous
import functools

import jax
import jax.numpy as jnp
from jax.experimental import pallas as pl
from jax.experimental.pallas import tpu as pltpu


def _gcn_kernel(adj_hbm, x_ref, w1_ref, b1_ref, w2_ref, b2_ref, out_ref,
                buf, h_ref, x16_ref, sems, *, bm, nchunk, slots):
    p = pl.program_id(0)
    i = pl.program_id(1)
    nsteps = pl.num_programs(1)
    g = p * nsteps + i
    ck = bm // nchunk

    def issue(step, slot):
        base = (step % nsteps) * bm
        for c in range(nchunk):
            pltpu.make_async_copy(
                adj_hbm.at[pl.ds(base + c * ck, ck), :],
                buf.at[slot, pl.ds(c * ck, ck), :],
                sems.at[slot],
            ).start()

    @pl.when(g == 0)
    def _():
        x16_ref[...] = x_ref[...].astype(jnp.bfloat16)
        for s in range(slots - 1):
            issue(s, s)

    @pl.when(g + slots - 1 < 2 * nsteps)
    def _():
        issue(g + slots - 1, (g + slots - 1) % slots)

    slot = g % slots
    for c in range(nchunk):
        pltpu.make_async_copy(
            adj_hbm.at[pl.ds(c * ck, ck), :],
            buf.at[slot, pl.ds(c * ck, ck), :],
            sems.at[slot],
        ).wait()

    a16 = buf[slot].astype(jnp.bfloat16)

    @pl.when(p == 0)
    def _():
        t = jnp.dot(a16, x16_ref[...], preferred_element_type=jnp.float32)
        t = jnp.dot(t.astype(jnp.bfloat16), w1_ref[...].astype(jnp.bfloat16),
                    preferred_element_type=jnp.float32) + b1_ref[...]
        h_ref[pl.ds(i * bm, bm), :] = jnp.maximum(t, 0.0).astype(jnp.bfloat16)

    @pl.when(p == 1)
    def _():
        t = jnp.dot(a16, h_ref[...], preferred_element_type=jnp.float32)
        t = jnp.dot(t.astype(jnp.bfloat16), w2_ref[...].astype(jnp.bfloat16),
                    preferred_element_type=jnp.float32) + b2_ref[...]
        out_ref[...] = t


def _gcn(x, adj, W1, b1, W2, b2, *, bm, nchunk, slots):
    n, k = adj.shape
    d = W1.shape[1]
    return pl.pallas_call(
        functools.partial(_gcn_kernel, bm=bm, nchunk=nchunk, slots=slots),
        grid=(2, n // bm),
        in_specs=[
            pl.BlockSpec(memory_space=pl.ANY),
            pl.BlockSpec((k, d), lambda p, i: (0, 0)),
            pl.BlockSpec((d, d), lambda p, i: (0, 0)),
            pl.BlockSpec((1, d), lambda p, i: (0, 0)),
            pl.BlockSpec((d, d), lambda p, i: (0, 0)),
            pl.BlockSpec((1, d), lambda p, i: (0, 0)),
        ],
        out_specs=pl.BlockSpec((bm, d), lambda p, i: (p * i, 0)),
        out_shape=jax.ShapeDtypeStruct((n, d), jnp.float32),
        compiler_params=pltpu.CompilerParams(
            vmem_limit_bytes=63 * 1024 * 1024),
        scratch_shapes=[
            pltpu.VMEM((slots, bm, k), jnp.float32),
            pltpu.VMEM((n, d), jnp.bfloat16),
            pltpu.VMEM((k, d), jnp.bfloat16),
            pltpu.SemaphoreType.DMA((slots,)),
        ],
    )(adj, x, W1, b1.reshape(1, d), W2, b2.reshape(1, d))


def kernel(x, adj, W1, b1, W2, b2):
    return _gcn(x, adj, W1, b1, W2, b2, bm=400, nchunk=1, slots=2)
